# baseline (speedup 1.0000x reference)
import functools

import jax
import jax.numpy as jnp
from jax import lax
from jax.experimental import pallas as pl
from jax.experimental.pallas import tpu as pltpu
from jax.experimental.pallas import tpu_sc as plsc

D_VOCAB = 100000
D_MODEL = 768
N_TOK = 8192
NC = 2
NS = 16
NW = NC * NS
ROWS_PER_WORKER = D_MODEL // NW
IW = 1024
NWIN = N_TOK // IW


def _gather_body(x_hbm, w_hbm, outT_hbm, idx_v, row_idx, val_v, sem):
    c = lax.axis_index("c")
    s = lax.axis_index("s")
    wid = s * NC + c

    pltpu.sync_copy(x_hbm, idx_v)

    def per_row(i, carry):
        d = wid * ROWS_PER_WORKER + i
        base = d * D_VOCAB

        def build(g, carry2):
            row_idx[pl.ds(g * 16, 16)] = idx_v[pl.ds(g * 16, 16)] + base
            return carry2

        lax.fori_loop(0, N_TOK // 16, build, 0, unroll=8)

        def fire(k, carry2):
            pltpu.async_copy(
                w_hbm.at[row_idx.at[pl.ds(k * IW, IW)]],
                val_v.at[pl.ds(k * IW, IW)],
                sem,
            )
            return carry2

        lax.fori_loop(0, NWIN, fire, 0)

        def drain(k, carry2):
            pltpu.make_async_copy(
                w_hbm.at[row_idx.at[pl.ds(k * IW, IW)]],
                val_v.at[pl.ds(k * IW, IW)],
                sem,
            ).wait()
            return carry2

        lax.fori_loop(0, NWIN, drain, 0)
        pltpu.sync_copy(val_v, outT_hbm.at[d])
        return carry

    lax.fori_loop(0, ROWS_PER_WORKER, per_row, 0)


@jax.jit
def _gather_rows(x_flat, w_flat):
    mesh = plsc.VectorSubcoreMesh(core_axis_name="c", subcore_axis_name="s")
    fn = functools.partial(
        pl.kernel,
        out_type=jax.ShapeDtypeStruct((D_MODEL, N_TOK), jnp.float32),
        mesh=mesh,
        scratch_types=[
            pltpu.VMEM((N_TOK,), jnp.int32),
            pltpu.VMEM((N_TOK,), jnp.int32),
            pltpu.VMEM((N_TOK,), jnp.float32),
            pltpu.SemaphoreType.DMA,
        ],
        compiler_params=pltpu.CompilerParams(
            needs_layout_passes=False,
            use_tc_tiling_on_sc=False,
        ),
    )(_gather_body)
    return fn(x_flat, w_flat)


def kernel(x, W_E):
    b, s = x.shape
    x_flat = x.reshape(-1).astype(jnp.int32)
    w_flat = W_E.reshape(-1)
    outT = _gather_rows(x_flat, w_flat)
    return jnp.transpose(outT).reshape(b, s, D_MODEL)


# speedup vs baseline: 1.6298x; 1.6298x over previous
import functools

import jax
import jax.numpy as jnp
from jax import lax
from jax.experimental import pallas as pl
from jax.experimental.pallas import tpu as pltpu
from jax.experimental.pallas import tpu_sc as plsc

D_VOCAB = 100000
D_MODEL = 768
N_TOK = 8192
NC = 2
NS = 16
NW = NC * NS
ROWS = D_MODEL // NW
HA = 50048
B0 = HA
HB = 49920
TAIL0 = B0 + HB
TAIL_W = 32
GV = N_TOK // 16


def _body(x_hbm, w_hbm, wt_hbm, outT_hbm, idx_v, buf_a, buf_b, tail_v,
          val_a, val_b, ssem, osems):
    c = lax.axis_index("c")
    s = lax.axis_index("s")
    wid = s * NC + c
    d_base = pl.multiple_of(wid * ROWS, 8)

    pltpu.sync_copy(x_hbm, idx_v)
    pltpu.sync_copy(wt_hbm.at[pl.ds(d_base, ROWS), :], tail_v)

    iz = lax.iota(jnp.int32, 16) * 0

    def cp_a(r):
        return (w_hbm.at[d_base + r, pl.ds(0, HA)], buf_a)

    def cp_b(r):
        return (w_hbm.at[d_base + r, pl.ds(B0, HB)], buf_b)

    def issue_a(r):
        pltpu.async_copy(*cp_a(r), ssem)

    def wait_a(r):
        pltpu.make_async_copy(*cp_a(r), ssem).wait()

    def issue_b(r):
        pltpu.async_copy(*cp_b(r), ssem)

    def wait_b(r):
        pltpu.make_async_copy(*cp_b(r), ssem).wait()

    def gather_a(val):
        def g(j, carry):
            iv = idx_v[pl.ds(j * 16, 16)]
            cl = jnp.minimum(iv, HA - 1)
            val[pl.ds(j * 16, 16)] = plsc.load_gather(buf_a, [cl])
            return carry

        lax.fori_loop(0, GV, g, 0, unroll=8)

    def gather_b(r, val):
        rvec = iz + r

        def g(j, carry):
            iv = idx_v[pl.ds(j * 16, 16)]
            cl = jnp.minimum(jnp.maximum(iv - B0, 0), HB - 1)
            gv = plsc.load_gather(buf_b, [cl])
            cl2 = jnp.maximum(iv - TAIL0, 0)
            tv = plsc.load_gather(tail_v, [rvec, cl2])
            cur = val[pl.ds(j * 16, 16)]
            res = jnp.where(iv >= B0, gv, cur)
            res = jnp.where(iv >= TAIL0, tv, res)
            val[pl.ds(j * 16, 16)] = res
            return carry

        lax.fori_loop(0, GV, g, 0, unroll=8)

    def out_cp(r, val):
        d = wid * ROWS + r
        return (val, outT_hbm.at[d])

    issue_a(0)

    def pair(p, carry):
        r0 = 2 * p
        r1 = r0 + 1

        wait_a(r0)
        issue_b(r0)

        @pl.when(p >= 1)
        def _():
            pltpu.make_async_copy(*out_cp(r0 - 2, val_a), osems.at[0]).wait()

        gather_a(val_a)
        wait_b(r0)
        issue_a(r1)
        gather_b(r0, val_a)
        pltpu.async_copy(*out_cp(r0, val_a), osems.at[0])

        wait_a(r1)
        issue_b(r1)

        @pl.when(p >= 1)
        def _():
            pltpu.make_async_copy(*out_cp(r1 - 2, val_b), osems.at[1]).wait()

        gather_a(val_b)
        wait_b(r1)

        @pl.when(p < ROWS // 2 - 1)
        def _():
            issue_a(r0 + 2)

        gather_b(r1, val_b)
        pltpu.async_copy(*out_cp(r1, val_b), osems.at[1])
        return carry

    lax.fori_loop(0, ROWS // 2, pair, 0)

    pltpu.make_async_copy(*out_cp(ROWS - 2, val_a), osems.at[0]).wait()
    pltpu.make_async_copy(*out_cp(ROWS - 1, val_b), osems.at[1]).wait()


@jax.jit
def _gather_rows(x_flat, w, w_tail):
    mesh = plsc.VectorSubcoreMesh(core_axis_name="c", subcore_axis_name="s")
    fn = functools.partial(
        pl.kernel,
        out_type=jax.ShapeDtypeStruct((D_MODEL, N_TOK), jnp.float32),
        mesh=mesh,
        scratch_types=[
            pltpu.VMEM((N_TOK,), jnp.int32),
            pltpu.VMEM((HA,), jnp.float32),
            pltpu.VMEM((HB,), jnp.float32),
            pltpu.VMEM((ROWS, TAIL_W), jnp.float32),
            pltpu.VMEM((N_TOK,), jnp.float32),
            pltpu.VMEM((N_TOK,), jnp.float32),
            pltpu.SemaphoreType.DMA,
            pltpu.SemaphoreType.DMA((2,)),
        ],
        compiler_params=pltpu.CompilerParams(needs_layout_passes=False),
    )(_body)
    return fn(x_flat, w, w_tail)


def kernel(x, W_E):
    b, s = x.shape
    x_flat = x.reshape(-1).astype(jnp.int32)
    w_tail = W_E[:, TAIL0:]
    outT = _gather_rows(x_flat, W_E, w_tail)
    return jnp.transpose(outT).reshape(b, s, D_MODEL)


# speedup vs baseline: 1.8913x; 1.1605x over previous
import functools

import jax
import jax.numpy as jnp
from jax import lax
from jax.experimental import pallas as pl
from jax.experimental.pallas import tpu as pltpu
from jax.experimental.pallas import tpu_sc as plsc

D_VOCAB = 100000
D_MODEL = 768
N_TOK = 8192
NUM_WORKERS = 32
ROWS_PER_WORKER = D_MODEL // NUM_WORKERS
LANES = 16


def _gather_body(x_hbm, w_hbm, outT_hbm, idx_v, row_v, val_v):
    c = lax.axis_index("c")
    s = lax.axis_index("s")
    wid = s * 2 + c

    pltpu.sync_copy(x_hbm, idx_v)

    def per_row(i, carry):
        d = wid * ROWS_PER_WORKER + i
        pltpu.sync_copy(w_hbm.at[d], row_v)

        def per_vec(j, carry2):
            iv = idx_v[pl.ds(j * LANES, LANES)]
            val_v[pl.ds(j * LANES, LANES)] = plsc.load_gather(row_v, [iv])
            return carry2

        lax.fori_loop(0, N_TOK // LANES, per_vec, 0, unroll=8)
        pltpu.sync_copy(val_v, outT_hbm.at[d])
        return carry

    lax.fori_loop(0, ROWS_PER_WORKER, per_row, 0)


@jax.jit
def _gather_rows(x_flat, w):
    mesh = plsc.VectorSubcoreMesh(core_axis_name="c", subcore_axis_name="s")
    fn = functools.partial(
        pl.kernel,
        out_type=jax.ShapeDtypeStruct((D_MODEL, N_TOK), jnp.float32),
        mesh=mesh,
        scratch_types=[
            pltpu.VMEM((N_TOK,), jnp.int32),
            pltpu.VMEM((D_VOCAB,), jnp.float32),
            pltpu.VMEM((N_TOK,), jnp.float32),
        ],
        compiler_params=pltpu.CompilerParams(needs_layout_passes=False),
    )(_gather_body)
    return fn(x_flat, w)


def kernel(x, W_E):
    b, s = x.shape
    x_flat = x.reshape(-1).astype(jnp.int32)
    outT = _gather_rows(x_flat, W_E)
    return jnp.transpose(outT).reshape(b, s, D_MODEL)


# speedup vs baseline: 1.9084x; 1.0090x over previous
import functools

import jax
import jax.numpy as jnp
from jax import lax
from jax.experimental import pallas as pl
from jax.experimental.pallas import tpu as pltpu
from jax.experimental.pallas import tpu_sc as plsc

D_VOCAB = 100000
D_MODEL = 768
N_TOK = 8192
NUM_WORKERS = 32
ROWS_PER_WORKER = D_MODEL // NUM_WORKERS
LANES = 16


def _gather_body(x_hbm, w_hbm, outT_hbm, idx_v, row_v, val_a, val_b, osems):
    c = lax.axis_index("c")
    s = lax.axis_index("s")
    wid = s * 2 + c

    pltpu.sync_copy(x_hbm, idx_v)

    def do_row(d, val):
        pltpu.sync_copy(w_hbm.at[d], row_v)

        def per_vec(j, carry2):
            iv = idx_v[pl.ds(j * LANES, LANES)]
            val[pl.ds(j * LANES, LANES)] = plsc.load_gather(row_v, [iv])
            return carry2

        lax.fori_loop(0, N_TOK // LANES, per_vec, 0, unroll=8)

    def per_pair(p, carry):
        d0 = wid * ROWS_PER_WORKER + 2 * p

        @pl.when(p >= 1)
        def _():
            pltpu.make_async_copy(
                val_a, outT_hbm.at[d0 - 2], osems.at[0]
            ).wait()

        do_row(d0, val_a)
        pltpu.async_copy(val_a, outT_hbm.at[d0], osems.at[0])

        @pl.when(p >= 1)
        def _():
            pltpu.make_async_copy(
                val_b, outT_hbm.at[d0 - 1], osems.at[1]
            ).wait()

        do_row(d0 + 1, val_b)
        pltpu.async_copy(val_b, outT_hbm.at[d0 + 1], osems.at[1])
        return carry

    lax.fori_loop(0, ROWS_PER_WORKER // 2, per_pair, 0)

    last = wid * ROWS_PER_WORKER + ROWS_PER_WORKER - 2
    pltpu.make_async_copy(val_a, outT_hbm.at[last], osems.at[0]).wait()
    pltpu.make_async_copy(val_b, outT_hbm.at[last + 1], osems.at[1]).wait()


@jax.jit
def _gather_rows(x_flat, w):
    mesh = plsc.VectorSubcoreMesh(core_axis_name="c", subcore_axis_name="s")
    fn = functools.partial(
        pl.kernel,
        out_type=jax.ShapeDtypeStruct((D_MODEL, N_TOK), jnp.float32),
        mesh=mesh,
        scratch_types=[
            pltpu.VMEM((N_TOK,), jnp.int32),
            pltpu.VMEM((D_VOCAB,), jnp.float32),
            pltpu.VMEM((N_TOK,), jnp.float32),
            pltpu.VMEM((N_TOK,), jnp.float32),
            pltpu.SemaphoreType.DMA((2,)),
        ],
        compiler_params=pltpu.CompilerParams(needs_layout_passes=False),
    )(_gather_body)
    return fn(x_flat, w)


def kernel(x, W_E):
    b, s = x.shape
    x_flat = x.reshape(-1).astype(jnp.int32)
    outT = _gather_rows(x_flat, W_E)
    return jnp.transpose(outT).reshape(b, s, D_MODEL)
